# baseline (speedup 1.0000x reference)
import functools

import jax
import jax.numpy as jnp
from jax import lax
from jax.experimental import pallas as pl
from jax.experimental.pallas import tpu as pltpu
from jax.experimental.pallas import tpu_sc as plsc

CONF_THRES = 0.25
IOU_THRES = 0.45
MAX_DET = 300

N = 20000
L = 16
NT = 16
PER = 1280
NPAD = NT * PER
CHUNKS = PER // L
OUT_ROWS = 304
BIG = 2**30


def _iota16():
    return lax.broadcasted_iota(jnp.int32, (L,), 0)


def _spf(x):
    return jnp.full((L,), x, jnp.float32)


def _spi(x):
    return jnp.full((L,), x, jnp.int32)


def _sc_body(x1h, y1h, x2h, y2h, sh, outh,
             x1v, y1v, x2v, y2v, wv, pubv, gv, obuf, pub_sp):
    cid = lax.axis_index("c")
    sid = lax.axis_index("s")

    @pl.when(cid == 0)
    def _run():
        base = sid * PER
        pltpu.sync_copy(x1h.at[pl.ds(base, PER)], x1v)
        pltpu.sync_copy(y1h.at[pl.ds(base, PER)], y1v)
        pltpu.sync_copy(x2h.at[pl.ds(base, PER)], x2v)
        pltpu.sync_copy(y2h.at[pl.ds(base, PER)], y2v)
        pltpu.sync_copy(sh.at[pl.ds(base, PER)], wv)
        iota = _iota16()
        run0 = jnp.full((L,), -jnp.inf, jnp.float32)

        @plsc.parallel_loop(0, CHUNKS, unroll=8, carry=(run0, iota))
        def init_loop(k, carry):
            run, idx = carry
            sl = pl.ds(k * L, L)
            x1c = x1v[sl]
            y1c = y1v[sl]
            x2c = x2v[sl]
            y2c = y2v[sl]
            s = wv[sl]
            w = jnp.where(s >= CONF_THRES, s, -jnp.inf)
            wv[sl] = w
            ci = k * L + iota
            take = (w > run) | ((w == run) & (ci < idx))
            run = jnp.where(take, w, run)
            idx = jnp.where(take, ci, idx)
            return run, idx

        run, idx = init_loop
        lv = jnp.max(run)
        li = jnp.min(jnp.where(run == lv, idx, BIG))

        def round_body(i, carry):
            lv, li = carry
            lidx = _spi(li)
            pubv[pl.ds(0, L)] = _spf(lv)
            pubv[pl.ds(L, L)] = plsc.bitcast(_spi(li + base), jnp.float32)
            pubv[pl.ds(2 * L, L)] = plsc.load_gather(x1v, [lidx])
            pubv[pl.ds(3 * L, L)] = plsc.load_gather(y1v, [lidx])
            pubv[pl.ds(4 * L, L)] = plsc.load_gather(x2v, [lidx])
            pubv[pl.ds(5 * L, L)] = plsc.load_gather(y2v, [lidx])
            slot = i & 1
            pltpu.sync_copy(pubv, pub_sp.at[slot, sid])
            plsc.subcore_barrier()
            pltpu.sync_copy(pub_sp.at[slot], gv)

            rows = iota
            mall = plsc.load_gather(gv, [rows, _spi(0)])
            iall = plsc.bitcast(plsc.load_gather(gv, [rows, _spi(L)]),
                                jnp.int32)
            gmax = jnp.max(mall)
            gj = jnp.min(jnp.where(mall == gmax, iall, BIG))
            rowsel = plsc.all_reduce_ffs(iall == gj)
            gmax16 = plsc.load_gather(gv, [rowsel, _spi(0)])
            x1g = plsc.load_gather(gv, [rowsel, _spi(2 * L)])
            y1g = plsc.load_gather(gv, [rowsel, _spi(3 * L)])
            x2g = plsc.load_gather(gv, [rowsel, _spi(4 * L)])
            y2g = plsc.load_gather(gv, [rowsel, _spi(5 * L)])

            @pl.when(sid == 0)
            def _write():
                fin16 = gmax16 > _spf(-jnp.inf)
                row = jnp.where(iota == 0, x1g,
                      jnp.where(iota == 1, y1g,
                      jnp.where(iota == 2, x2g,
                      jnp.where(iota == 3, y2g,
                      jnp.where(iota == 4, gmax16, _spf(0.0))))))
                obuf[pl.ds(i * L, L)] = jnp.where(fin16, row, _spf(0.0))

            area1 = (x2g - x1g) * (y2g - y1g)

            @plsc.parallel_loop(0, CHUNKS, unroll=8, carry=(run0, iota))
            def supp_loop(k, carry2):
                run, idx = carry2
                sl = pl.ds(k * L, L)
                x1c = x1v[sl]
                y1c = y1v[sl]
                x2c = x2v[sl]
                y2c = y2v[sl]
                w = wv[sl]
                ltx = jnp.maximum(x1g, x1c)
                lty = jnp.maximum(y1g, y1c)
                rbx = jnp.minimum(x2g, x2c)
                rby = jnp.minimum(y2g, y2c)
                iw = jnp.maximum(rbx - ltx, 0.0)
                ih = jnp.maximum(rby - lty, 0.0)
                inter = iw * ih
                a2c = (x2c - x1c) * (y2c - y1c)
                iou = inter / (area1 + a2c - inter + 1e-9)
                ci = k * L + iota
                w2 = jnp.where((iou > IOU_THRES) | ((base + ci) == gj),
                               -jnp.inf, w)
                wv[sl] = w2
                take = (w2 > run) | ((w2 == run) & (ci < idx))
                run = jnp.where(take, w2, run)
                idx = jnp.where(take, ci, idx)
                return run, idx

            run, idx = supp_loop
            lv2 = jnp.max(run)
            li2 = jnp.min(jnp.where(run == lv2, idx, BIG))
            return lv2, li2

        lax.fori_loop(0, MAX_DET, round_body, (lv, li))

        @pl.when(sid == 0)
        def _finish():
            pltpu.sync_copy(obuf, outh)


@jax.jit
def kernel(boxes, scores):
    bp = jnp.pad(boxes, ((0, NPAD - N), (0, 0)))
    sp = jnp.pad(scores, (0, NPAD - N))
    mesh = plsc.VectorSubcoreMesh(core_axis_name="c", subcore_axis_name="s",
                                  num_cores=2, num_subcores=NT)
    call = pl.kernel(
        _sc_body,
        out_type=jax.ShapeDtypeStruct((OUT_ROWS * L,), jnp.float32),
        mesh=mesh,
        compiler_params=pltpu.CompilerParams(needs_layout_passes=False),
        scratch_types=[
            pltpu.VMEM((PER,), jnp.float32),
            pltpu.VMEM((PER,), jnp.float32),
            pltpu.VMEM((PER,), jnp.float32),
            pltpu.VMEM((PER,), jnp.float32),
            pltpu.VMEM((PER,), jnp.float32),
            pltpu.VMEM((8 * L,), jnp.float32),
            pltpu.VMEM((NT, 8 * L), jnp.float32),
            pltpu.VMEM((OUT_ROWS * L,), jnp.float32),
            pltpu.VMEM_SHARED((2, NT, 8 * L), jnp.float32),
        ],
    )
    out = call(bp[:, 0], bp[:, 1], bp[:, 2], bp[:, 3], sp)
    return out.reshape(OUT_ROWS, L)[:MAX_DET, :5]


# speedup vs baseline: 1.0532x; 1.0532x over previous
import functools

import jax
import jax.numpy as jnp
from jax import lax
from jax.experimental import pallas as pl
from jax.experimental.pallas import tpu as pltpu
from jax.experimental.pallas import tpu_sc as plsc

CONF_THRES = 0.25
IOU_THRES = 0.45
MAX_DET = 300

N = 20000
L = 16
NT = 16
PER = 1280
NPAD = NT * PER
CHUNKS = PER // L
OUT_ROWS = 304
BIG = 2**30


def _iota16():
    return lax.broadcasted_iota(jnp.int32, (L,), 0)


def _spf(x):
    return jnp.full((L,), x, jnp.float32)


def _spi(x):
    return jnp.full((L,), x, jnp.int32)


def _sc_body(x1h, y1h, x2h, y2h, sh, outh,
             x1v, y1v, x2v, y2v, wv, pubv, gv, obuf, pub_sp):
    cid = lax.axis_index("c")
    sid = lax.axis_index("s")

    @pl.when(cid == 0)
    def _run():
        base = sid * PER
        pltpu.sync_copy(x1h.at[pl.ds(base, PER)], x1v)
        pltpu.sync_copy(y1h.at[pl.ds(base, PER)], y1v)
        pltpu.sync_copy(x2h.at[pl.ds(base, PER)], x2v)
        pltpu.sync_copy(y2h.at[pl.ds(base, PER)], y2v)
        pltpu.sync_copy(sh.at[pl.ds(base, PER)], wv)
        iota = _iota16()
        run0 = jnp.full((L,), -jnp.inf, jnp.float32)

        @plsc.parallel_loop(0, CHUNKS, unroll=4, carry=(run0, iota))
        def init_loop(k, carry):
            run, idx = carry
            sl = pl.ds(k * L, L)
            x1c = x1v[sl]
            y1c = y1v[sl]
            x2c = x2v[sl]
            y2c = y2v[sl]
            s = wv[sl]
            w = jnp.where(s >= CONF_THRES, s, -jnp.inf)
            wv[sl] = w
            ci = k * L + iota
            take = (w > run) | ((w == run) & (ci < idx))
            run = jnp.where(take, w, run)
            idx = jnp.where(take, ci, idx)
            return run, idx

        run, idx = init_loop
        lv = jnp.max(run)
        li = jnp.min(jnp.where(run == lv, idx, BIG))

        def round_body(i, carry):
            lv, li = carry
            lidx = _spi(li)
            pubv[pl.ds(0, L)] = _spf(lv)
            pubv[pl.ds(L, L)] = plsc.bitcast(_spi(li + base), jnp.float32)
            pubv[pl.ds(2 * L, L)] = plsc.load_gather(x1v, [lidx])
            pubv[pl.ds(3 * L, L)] = plsc.load_gather(y1v, [lidx])
            pubv[pl.ds(4 * L, L)] = plsc.load_gather(x2v, [lidx])
            pubv[pl.ds(5 * L, L)] = plsc.load_gather(y2v, [lidx])
            slot = i & 1
            pltpu.sync_copy(pubv, pub_sp.at[slot, sid])
            plsc.subcore_barrier()
            pltpu.sync_copy(pub_sp.at[slot], gv)

            rows = iota
            mall = plsc.load_gather(gv, [rows, _spi(0)])
            iall = plsc.bitcast(plsc.load_gather(gv, [rows, _spi(L)]),
                                jnp.int32)
            gmax = jnp.max(mall)
            gj = jnp.min(jnp.where(mall == gmax, iall, BIG))
            rowsel = plsc.all_reduce_ffs(iall == gj)
            gmax16 = plsc.load_gather(gv, [rowsel, _spi(0)])
            x1g = plsc.load_gather(gv, [rowsel, _spi(2 * L)])
            y1g = plsc.load_gather(gv, [rowsel, _spi(3 * L)])
            x2g = plsc.load_gather(gv, [rowsel, _spi(4 * L)])
            y2g = plsc.load_gather(gv, [rowsel, _spi(5 * L)])

            @pl.when(sid == 0)
            def _write():
                fin16 = gmax16 > _spf(-jnp.inf)
                row = jnp.where(iota == 0, x1g,
                      jnp.where(iota == 1, y1g,
                      jnp.where(iota == 2, x2g,
                      jnp.where(iota == 3, y2g,
                      jnp.where(iota == 4, gmax16, _spf(0.0))))))
                obuf[pl.ds(i * L, L)] = jnp.where(fin16, row, _spf(0.0))

            area1 = (x2g - x1g) * (y2g - y1g)

            @plsc.parallel_loop(0, CHUNKS, unroll=4, carry=(run0, iota))
            def supp_loop(k, carry2):
                run, idx = carry2
                sl = pl.ds(k * L, L)
                x1c = x1v[sl]
                y1c = y1v[sl]
                x2c = x2v[sl]
                y2c = y2v[sl]
                w = wv[sl]
                ltx = jnp.maximum(x1g, x1c)
                lty = jnp.maximum(y1g, y1c)
                rbx = jnp.minimum(x2g, x2c)
                rby = jnp.minimum(y2g, y2c)
                iw = jnp.maximum(rbx - ltx, 0.0)
                ih = jnp.maximum(rby - lty, 0.0)
                inter = iw * ih
                a2c = (x2c - x1c) * (y2c - y1c)
                iou = inter / (area1 + a2c - inter + 1e-9)
                ci = k * L + iota
                w2 = jnp.where((iou > IOU_THRES) | ((base + ci) == gj),
                               -jnp.inf, w)
                wv[sl] = w2
                take = (w2 > run) | ((w2 == run) & (ci < idx))
                run = jnp.where(take, w2, run)
                idx = jnp.where(take, ci, idx)
                return run, idx

            run, idx = supp_loop
            lv2 = jnp.max(run)
            li2 = jnp.min(jnp.where(run == lv2, idx, BIG))
            return lv2, li2

        lax.fori_loop(0, MAX_DET, round_body, (lv, li))

        @pl.when(sid == 0)
        def _finish():
            pltpu.sync_copy(obuf, outh)


@jax.jit
def kernel(boxes, scores):
    bp = jnp.pad(boxes, ((0, NPAD - N), (0, 0)))
    sp = jnp.pad(scores, (0, NPAD - N))
    mesh = plsc.VectorSubcoreMesh(core_axis_name="c", subcore_axis_name="s",
                                  num_cores=2, num_subcores=NT)
    call = pl.kernel(
        _sc_body,
        out_type=jax.ShapeDtypeStruct((OUT_ROWS * L,), jnp.float32),
        mesh=mesh,
        compiler_params=pltpu.CompilerParams(needs_layout_passes=False),
        scratch_types=[
            pltpu.VMEM((PER,), jnp.float32),
            pltpu.VMEM((PER,), jnp.float32),
            pltpu.VMEM((PER,), jnp.float32),
            pltpu.VMEM((PER,), jnp.float32),
            pltpu.VMEM((PER,), jnp.float32),
            pltpu.VMEM((8 * L,), jnp.float32),
            pltpu.VMEM((NT, 8 * L), jnp.float32),
            pltpu.VMEM((OUT_ROWS * L,), jnp.float32),
            pltpu.VMEM_SHARED((2, NT, 8 * L), jnp.float32),
        ],
    )
    out = call(bp[:, 0], bp[:, 1], bp[:, 2], bp[:, 3], sp)
    return out.reshape(OUT_ROWS, L)[:MAX_DET, :5]


# speedup vs baseline: 1.1308x; 1.0737x over previous
import jax
import jax.numpy as jnp
from jax import lax
from jax.experimental import pallas as pl
from jax.experimental.pallas import tpu as pltpu
from jax.experimental.pallas import tpu_sc as plsc

CONF_THRES = 0.25
IOU_THRES = 0.45
MAX_DET = 300

N = 20000
L = 16
NT = 16
PER = 1280
NPAD = NT * PER
CHUNKS = PER // L
OUT_ROWS = 304
BIG = 2**30
ROW = 128


def _iota16():
    return lax.broadcasted_iota(jnp.int32, (L,), 0)


def _spf(x):
    return jnp.full((L,), x, jnp.float32)


def _spi(x):
    return jnp.full((L,), x, jnp.int32)


def _iou16(ax1, ay1, ax2, ay2, bx1, by1, bx2, by2, area_a):
    ltx = jnp.maximum(ax1, bx1)
    lty = jnp.maximum(ay1, by1)
    rbx = jnp.minimum(ax2, bx2)
    rby = jnp.minimum(ay2, by2)
    iw = jnp.maximum(rbx - ltx, 0.0)
    ih = jnp.maximum(rby - lty, 0.0)
    inter = iw * ih
    area_b = (bx2 - bx1) * (by2 - by1)
    return inter / (area_a + area_b - inter + 1e-9)


def _sc_body(x1h, y1h, x2h, y2h, sh, outh,
             x1v, y1v, x2v, y2v, wv, pubv, gv, obuf, pub_sp):
    cid = lax.axis_index("c")
    sid = lax.axis_index("s")

    @pl.when(cid == 0)
    def _run():
        base = sid * PER
        pltpu.sync_copy(x1h.at[pl.ds(base, PER)], x1v)
        pltpu.sync_copy(y1h.at[pl.ds(base, PER)], y1v)
        pltpu.sync_copy(x2h.at[pl.ds(base, PER)], x2v)
        pltpu.sync_copy(y2h.at[pl.ds(base, PER)], y2v)
        pltpu.sync_copy(sh.at[pl.ds(base, PER)], wv)
        iota = _iota16()
        ninf = jnp.full((L,), -jnp.inf, jnp.float32)

        def top2_insert(r1, i1, r2, i2, w, ci):
            above1 = (w > r1) | ((w == r1) & (ci < i1))
            nr1 = jnp.where(above1, w, r1)
            ni1 = jnp.where(above1, ci, i1)
            dv = jnp.where(above1, r1, w)
            di = jnp.where(above1, i1, ci)
            above2 = (dv > r2) | ((dv == r2) & (di < i2))
            nr2 = jnp.where(above2, dv, r2)
            ni2 = jnp.where(above2, di, i2)
            return nr1, ni1, nr2, ni2

        @plsc.parallel_loop(0, CHUNKS, unroll=4,
                            carry=(ninf, iota, ninf, iota))
        def init_loop(k, carry):
            r1, i1, r2, i2 = carry
            sl = pl.ds(k * L, L)
            s = wv[sl]
            w = jnp.where(s >= CONF_THRES, s, -jnp.inf)
            wv[sl] = w
            return top2_insert(r1, i1, r2, i2, w, k * L + iota)

        r1, i1, r2, i2 = init_loop

        def publish(r1, i1, r2, i2):
            m1 = jnp.max(r1)
            l1 = jnp.min(jnp.where(r1 == m1, i1, BIG))
            winlane = i1 == l1
            c = jnp.where(winlane, r2, r1)
            cidx = jnp.where(winlane, i2, i1)
            m2 = jnp.max(c)
            l2 = jnp.min(jnp.where(c == m2, cidx, BIG))
            for off, m, li in ((0, m1, l1), (ROW, m2, l2)):
                lidx = _spi(li)
                pubv[pl.ds(off, L)] = _spf(m)
                pubv[pl.ds(off + L, L)] = plsc.bitcast(
                    _spi(li + base), jnp.float32)
                pubv[pl.ds(off + 2 * L, L)] = plsc.load_gather(x1v, [lidx])
                pubv[pl.ds(off + 3 * L, L)] = plsc.load_gather(y1v, [lidx])
                pubv[pl.ds(off + 4 * L, L)] = plsc.load_gather(x2v, [lidx])
                pubv[pl.ds(off + 5 * L, L)] = plsc.load_gather(y2v, [lidx])

        publish(r1, i1, r2, i2)

        def round_body(carry):
            oi, it = carry
            slot = it & 1
            pltpu.sync_copy(pubv, pub_sp.at[slot, sid])
            plsc.subcore_barrier()
            pltpu.sync_copy(pub_sp.at[slot], gv)

            rows = iota
            mall1 = plsc.load_gather(gv, [rows, _spi(0)])
            iall1 = plsc.bitcast(plsc.load_gather(gv, [rows, _spi(L)]),
                                 jnp.int32)
            mall2 = plsc.load_gather(gv, [rows, _spi(ROW)])
            iall2 = plsc.bitcast(plsc.load_gather(gv, [rows, _spi(ROW + L)]),
                                 jnp.int32)

            gmax1 = jnp.max(mall1)
            gj1 = jnp.min(jnp.where(mall1 == gmax1, iall1, BIG))
            sel1 = iall1 == gj1
            row1 = plsc.all_reduce_ffs(sel1)
            gm1 = plsc.load_gather(gv, [row1, _spi(0)])
            x1a = plsc.load_gather(gv, [row1, _spi(2 * L)])
            y1a = plsc.load_gather(gv, [row1, _spi(3 * L)])
            x2a = plsc.load_gather(gv, [row1, _spi(4 * L)])
            y2a = plsc.load_gather(gv, [row1, _spi(5 * L)])

            c2 = jnp.where(sel1, mall2, mall1)
            ci2 = jnp.where(sel1, iall2, iall1)
            cs2 = jnp.where(sel1, ROW, 0)
            gmax2 = jnp.max(c2)
            gj2 = jnp.min(jnp.where(c2 == gmax2, ci2, BIG))
            sel2 = ci2 == gj2
            row2 = plsc.all_reduce_ffs(sel2)
            s2 = jnp.min(jnp.where(sel2, cs2, BIG))
            s2v = _spi(s2)
            gm2 = plsc.load_gather(gv, [row2, s2v])
            x1b = plsc.load_gather(gv, [row2, s2v + 2 * L])
            y1b = plsc.load_gather(gv, [row2, s2v + 3 * L])
            x2b = plsc.load_gather(gv, [row2, s2v + 4 * L])
            y2b = plsc.load_gather(gv, [row2, s2v + 5 * L])

            area_a = (x2a - x1a) * (y2a - y1a)
            iou12 = _iou16(x1a, y1a, x2a, y2a, x1b, y1b, x2b, y2b, area_a)
            bad2 = jnp.max(jnp.where(iou12 > IOU_THRES, 1, 0))
            valid2 = bad2 == 0

            @pl.when(sid == 0)
            def _write():
                fin1 = gm1 > _spf(-jnp.inf)
                rowv = jnp.where(iota == 0, x1a,
                       jnp.where(iota == 1, y1a,
                       jnp.where(iota == 2, x2a,
                       jnp.where(iota == 3, y2a,
                       jnp.where(iota == 4, gm1, _spf(0.0))))))
                obuf[pl.ds(oi * L, L)] = jnp.where(fin1, rowv, _spf(0.0))

                @pl.when(valid2)
                def _write2():
                    fin2 = gm2 > _spf(-jnp.inf)
                    rw2 = jnp.where(iota == 0, x1b,
                          jnp.where(iota == 1, y1b,
                          jnp.where(iota == 2, x2b,
                          jnp.where(iota == 3, y2b,
                          jnp.where(iota == 4, gm2, _spf(0.0))))))
                    obuf[pl.ds((oi + 1) * L, L)] = jnp.where(
                        fin2, rw2, _spf(0.0))

            v2f = jnp.where(valid2, _spf(1.0), _spf(0.0))
            ex1b = x1b * v2f
            ey1b = y1b * v2f
            ex2b = x2b * v2f
            ey2b = y2b * v2f
            area_b2 = (ex2b - ex1b) * (ey2b - ey1b)
            gj2e = jnp.where(valid2, gj2, -1)

            @plsc.parallel_loop(0, CHUNKS, unroll=4,
                                carry=(ninf, iota, ninf, iota))
            def supp_loop(k, carry2):
                r1, i1, r2, i2 = carry2
                sl = pl.ds(k * L, L)
                x1c = x1v[sl]
                y1c = y1v[sl]
                x2c = x2v[sl]
                y2c = y2v[sl]
                w = wv[sl]
                a2c = (x2c - x1c) * (y2c - y1c)
                ltx = jnp.maximum(x1a, x1c)
                lty = jnp.maximum(y1a, y1c)
                rbx = jnp.minimum(x2a, x2c)
                rby = jnp.minimum(y2a, y2c)
                iw = jnp.maximum(rbx - ltx, 0.0)
                ih = jnp.maximum(rby - lty, 0.0)
                inter = iw * ih
                iou1 = inter / (area_a + a2c - inter + 1e-9)
                ltx2 = jnp.maximum(ex1b, x1c)
                lty2 = jnp.maximum(ey1b, y1c)
                rbx2 = jnp.minimum(ex2b, x2c)
                rby2 = jnp.minimum(ey2b, y2c)
                iw2 = jnp.maximum(rbx2 - ltx2, 0.0)
                ih2 = jnp.maximum(rby2 - lty2, 0.0)
                inter2 = iw2 * ih2
                iou2 = inter2 / (area_b2 + a2c - inter2 + 1e-9)
                ci = k * L + iota
                g = base + ci
                kill = (iou1 > IOU_THRES) | (iou2 > IOU_THRES) \
                    | (g == gj1) | (g == gj2e)
                w2w = jnp.where(kill, -jnp.inf, w)
                wv[sl] = w2w
                return top2_insert(r1, i1, r2, i2, w2w, ci)

            nr1, ni1, nr2, ni2 = supp_loop
            publish(nr1, ni1, nr2, ni2)
            oi_next = oi + jnp.where(valid2, 2, 1)
            return oi_next, it + 1

        lax.while_loop(lambda c: c[0] < MAX_DET, round_body,
                       (jnp.int32(0), jnp.int32(0)))

        @pl.when(sid == 0)
        def _finish():
            pltpu.sync_copy(obuf, outh)


@jax.jit
def kernel(boxes, scores):
    bp = jnp.pad(boxes, ((0, NPAD - N), (0, 0)))
    sp = jnp.pad(scores, (0, NPAD - N))
    mesh = plsc.VectorSubcoreMesh(core_axis_name="c", subcore_axis_name="s",
                                  num_cores=2, num_subcores=NT)
    call = pl.kernel(
        _sc_body,
        out_type=jax.ShapeDtypeStruct((OUT_ROWS * L,), jnp.float32),
        mesh=mesh,
        compiler_params=pltpu.CompilerParams(needs_layout_passes=False),
        scratch_types=[
            pltpu.VMEM((PER,), jnp.float32),
            pltpu.VMEM((PER,), jnp.float32),
            pltpu.VMEM((PER,), jnp.float32),
            pltpu.VMEM((PER,), jnp.float32),
            pltpu.VMEM((PER,), jnp.float32),
            pltpu.VMEM((2 * ROW,), jnp.float32),
            pltpu.VMEM((NT, 2 * ROW), jnp.float32),
            pltpu.VMEM((OUT_ROWS * L,), jnp.float32),
            pltpu.VMEM_SHARED((2, NT, 2 * ROW), jnp.float32),
        ],
    )
    out = call(bp[:, 0], bp[:, 1], bp[:, 2], bp[:, 3], sp)
    return out.reshape(OUT_ROWS, L)[:MAX_DET, :5]


# speedup vs baseline: 1.2114x; 1.0713x over previous
import jax
import jax.numpy as jnp
from jax import lax
from jax.experimental import pallas as pl
from jax.experimental.pallas import tpu as pltpu
from jax.experimental.pallas import tpu_sc as plsc

CONF_THRES = 0.25
IOU_THRES = 0.45
MAX_DET = 300

N = 20000
L = 16
NT = 16
PER = 1280
NPAD = NT * PER
CHUNKS = PER // L
OUT_ROWS = 304
BIG = 2**30
ROW = 128


def _iota16():
    return lax.broadcasted_iota(jnp.int32, (L,), 0)


def _spf(x):
    return jnp.full((L,), x, jnp.float32)


def _spi(x):
    return jnp.full((L,), x, jnp.int32)


def _iou16(ax1, ay1, ax2, ay2, bx1, by1, bx2, by2, area_a):
    ltx = jnp.maximum(ax1, bx1)
    lty = jnp.maximum(ay1, by1)
    rbx = jnp.minimum(ax2, bx2)
    rby = jnp.minimum(ay2, by2)
    iw = jnp.maximum(rbx - ltx, 0.0)
    ih = jnp.maximum(rby - lty, 0.0)
    inter = iw * ih
    area_b = (bx2 - bx1) * (by2 - by1)
    return inter / (area_a + area_b - inter + 1e-9)


def _sc_body(x1h, y1h, x2h, y2h, sh, outh,
             x1v, y1v, x2v, y2v, wv, pubv, gv, obuf, pub_sp):
    cid = lax.axis_index("c")
    sid = lax.axis_index("s")

    @pl.when(cid == 0)
    def _run():
        base = sid * PER
        pltpu.sync_copy(x1h.at[pl.ds(base, PER)], x1v)
        pltpu.sync_copy(y1h.at[pl.ds(base, PER)], y1v)
        pltpu.sync_copy(x2h.at[pl.ds(base, PER)], x2v)
        pltpu.sync_copy(y2h.at[pl.ds(base, PER)], y2v)
        pltpu.sync_copy(sh.at[pl.ds(base, PER)], wv)
        iota = _iota16()
        ninf = jnp.full((L,), -jnp.inf, jnp.float32)

        def top2_insert(r1, i1, r2, i2, w, ci):
            above1 = (w > r1) | ((w == r1) & (ci < i1))
            nr1 = jnp.where(above1, w, r1)
            ni1 = jnp.where(above1, ci, i1)
            dv = jnp.where(above1, r1, w)
            di = jnp.where(above1, i1, ci)
            above2 = (dv > r2) | ((dv == r2) & (di < i2))
            nr2 = jnp.where(above2, dv, r2)
            ni2 = jnp.where(above2, di, i2)
            return nr1, ni1, nr2, ni2

        @plsc.parallel_loop(0, CHUNKS, unroll=4,
                            carry=(ninf, iota, ninf, iota))
        def init_loop(k, carry):
            r1, i1, r2, i2 = carry
            sl = pl.ds(k * L, L)
            s = wv[sl]
            w = jnp.where(s >= CONF_THRES, s, -jnp.inf)
            wv[sl] = w
            return top2_insert(r1, i1, r2, i2, w, k * L + iota)

        r1, i1, r2, i2 = init_loop

        def publish(r1, i1, r2, i2):
            m1 = jnp.max(r1)
            l1 = jnp.min(jnp.where(r1 == m1, i1, BIG))
            winlane = i1 == l1
            c = jnp.where(winlane, r2, r1)
            cidx = jnp.where(winlane, i2, i1)
            m2 = jnp.max(c)
            l2 = jnp.min(jnp.where(c == m2, cidx, BIG))
            for off, m, li in ((0, m1, l1), (ROW, m2, l2)):
                lidx = _spi(li)
                pubv[pl.ds(off, L)] = _spf(m)
                pubv[pl.ds(off + L, L)] = plsc.bitcast(
                    _spi(li + base), jnp.float32)
                pubv[pl.ds(off + 2 * L, L)] = plsc.load_gather(x1v, [lidx])
                pubv[pl.ds(off + 3 * L, L)] = plsc.load_gather(y1v, [lidx])
                pubv[pl.ds(off + 4 * L, L)] = plsc.load_gather(x2v, [lidx])
                pubv[pl.ds(off + 5 * L, L)] = plsc.load_gather(y2v, [lidx])

        publish(r1, i1, r2, i2)

        def round_body(carry):
            oi, it = carry
            slot = it & 1
            pltpu.sync_copy(pubv, pub_sp.at[slot, sid])
            plsc.subcore_barrier()
            pltpu.sync_copy(pub_sp.at[slot], gv)

            rows = iota
            mall1 = plsc.load_gather(gv, [rows, _spi(0)])
            iall1 = plsc.bitcast(plsc.load_gather(gv, [rows, _spi(L)]),
                                 jnp.int32)
            mall2 = plsc.load_gather(gv, [rows, _spi(ROW)])
            iall2 = plsc.bitcast(plsc.load_gather(gv, [rows, _spi(ROW + L)]),
                                 jnp.int32)

            gmax1 = jnp.max(mall1)
            gj1 = jnp.min(jnp.where(mall1 == gmax1, iall1, BIG))
            sel1 = iall1 == gj1
            row1 = plsc.all_reduce_ffs(sel1)
            gm1 = plsc.load_gather(gv, [row1, _spi(0)])
            x1a = plsc.load_gather(gv, [row1, _spi(2 * L)])
            y1a = plsc.load_gather(gv, [row1, _spi(3 * L)])
            x2a = plsc.load_gather(gv, [row1, _spi(4 * L)])
            y2a = plsc.load_gather(gv, [row1, _spi(5 * L)])

            c2 = jnp.where(sel1, mall2, mall1)
            ci2 = jnp.where(sel1, iall2, iall1)
            cs2 = jnp.where(sel1, ROW, 0)
            gmax2 = jnp.max(c2)
            gj2 = jnp.min(jnp.where(c2 == gmax2, ci2, BIG))
            sel2 = ci2 == gj2
            row2 = plsc.all_reduce_ffs(sel2)
            s2 = jnp.min(jnp.where(sel2, cs2, BIG))
            s2v = _spi(s2)
            gm2 = plsc.load_gather(gv, [row2, s2v])
            x1b = plsc.load_gather(gv, [row2, s2v + 2 * L])
            y1b = plsc.load_gather(gv, [row2, s2v + 3 * L])
            x2b = plsc.load_gather(gv, [row2, s2v + 4 * L])
            y2b = plsc.load_gather(gv, [row2, s2v + 5 * L])

            area_a = (x2a - x1a) * (y2a - y1a)
            iou12 = _iou16(x1a, y1a, x2a, y2a, x1b, y1b, x2b, y2b, area_a)
            bad2 = jnp.max(jnp.where(iou12 > IOU_THRES, 1, 0))
            valid2 = bad2 == 0

            @pl.when(sid == 0)
            def _write():
                fin1 = gm1 > _spf(-jnp.inf)
                rowv = jnp.where(iota == 0, x1a,
                       jnp.where(iota == 1, y1a,
                       jnp.where(iota == 2, x2a,
                       jnp.where(iota == 3, y2a,
                       jnp.where(iota == 4, gm1, _spf(0.0))))))
                obuf[pl.ds(oi * L, L)] = jnp.where(fin1, rowv, _spf(0.0))

                @pl.when(valid2)
                def _write2():
                    fin2 = gm2 > _spf(-jnp.inf)
                    rw2 = jnp.where(iota == 0, x1b,
                          jnp.where(iota == 1, y1b,
                          jnp.where(iota == 2, x2b,
                          jnp.where(iota == 3, y2b,
                          jnp.where(iota == 4, gm2, _spf(0.0))))))
                    obuf[pl.ds((oi + 1) * L, L)] = jnp.where(
                        fin2, rw2, _spf(0.0))

            v2f = jnp.where(valid2, _spf(1.0), _spf(0.0))
            ex1b = x1b * v2f
            ey1b = y1b * v2f
            ex2b = x2b * v2f
            ey2b = y2b * v2f
            area_b2 = (ex2b - ex1b) * (ey2b - ey1b)
            gj2e = jnp.where(valid2, gj2, -1)

            @plsc.parallel_loop(0, CHUNKS, unroll=2,
                                carry=(ninf, iota, ninf, iota))
            def supp_loop(k, carry2):
                r1, i1, r2, i2 = carry2
                sl = pl.ds(k * L, L)
                x1c = x1v[sl]
                y1c = y1v[sl]
                x2c = x2v[sl]
                y2c = y2v[sl]
                w = wv[sl]
                a2c = (x2c - x1c) * (y2c - y1c)
                ltx = jnp.maximum(x1a, x1c)
                lty = jnp.maximum(y1a, y1c)
                rbx = jnp.minimum(x2a, x2c)
                rby = jnp.minimum(y2a, y2c)
                iw = jnp.maximum(rbx - ltx, 0.0)
                ih = jnp.maximum(rby - lty, 0.0)
                inter = iw * ih
                iou1 = inter / (area_a + a2c - inter + 1e-9)
                ltx2 = jnp.maximum(ex1b, x1c)
                lty2 = jnp.maximum(ey1b, y1c)
                rbx2 = jnp.minimum(ex2b, x2c)
                rby2 = jnp.minimum(ey2b, y2c)
                iw2 = jnp.maximum(rbx2 - ltx2, 0.0)
                ih2 = jnp.maximum(rby2 - lty2, 0.0)
                inter2 = iw2 * ih2
                iou2 = inter2 / (area_b2 + a2c - inter2 + 1e-9)
                ci = k * L + iota
                g = base + ci
                kill = (iou1 > IOU_THRES) | (iou2 > IOU_THRES) \
                    | (g == gj1) | (g == gj2e)
                w2w = jnp.where(kill, -jnp.inf, w)
                wv[sl] = w2w
                return top2_insert(r1, i1, r2, i2, w2w, ci)

            nr1, ni1, nr2, ni2 = supp_loop
            publish(nr1, ni1, nr2, ni2)
            oi_next = oi + jnp.where(valid2, 2, 1)
            return oi_next, it + 1

        lax.while_loop(lambda c: c[0] < MAX_DET, round_body,
                       (jnp.int32(0), jnp.int32(0)))

        @pl.when(sid == 0)
        def _finish():
            pltpu.sync_copy(obuf, outh)


@jax.jit
def kernel(boxes, scores):
    bp = jnp.pad(boxes, ((0, NPAD - N), (0, 0)))
    sp = jnp.pad(scores, (0, NPAD - N))
    mesh = plsc.VectorSubcoreMesh(core_axis_name="c", subcore_axis_name="s",
                                  num_cores=2, num_subcores=NT)
    call = pl.kernel(
        _sc_body,
        out_type=jax.ShapeDtypeStruct((OUT_ROWS * L,), jnp.float32),
        mesh=mesh,
        compiler_params=pltpu.CompilerParams(needs_layout_passes=False),
        scratch_types=[
            pltpu.VMEM((PER,), jnp.float32),
            pltpu.VMEM((PER,), jnp.float32),
            pltpu.VMEM((PER,), jnp.float32),
            pltpu.VMEM((PER,), jnp.float32),
            pltpu.VMEM((PER,), jnp.float32),
            pltpu.VMEM((2 * ROW,), jnp.float32),
            pltpu.VMEM((NT, 2 * ROW), jnp.float32),
            pltpu.VMEM((OUT_ROWS * L,), jnp.float32),
            pltpu.VMEM_SHARED((2, NT, 2 * ROW), jnp.float32),
        ],
    )
    out = call(bp[:, 0], bp[:, 1], bp[:, 2], bp[:, 3], sp)
    return out.reshape(OUT_ROWS, L)[:MAX_DET, :5]
